# baseline (speedup 1.0000x reference)
import functools

import jax
import jax.numpy as jnp
from jax import lax
from jax.experimental import pallas as pl
from jax.experimental.pallas import tpu as pltpu
from jax.experimental.pallas import tpu_sc as plsc

N = 4096
K = 16
G = 8
H = 32
NCLS = 40
RB = 256
WLEN = 1408
R3 = 512
GW = 128
BIG = 1e30


def _knn_proj_body(posb_ref, posT_ref, bcol_ref, brow_ref, w1h_ref, w1p_ref,
                   w2p_ref, idx_ref, a1_ref, b1_ref, b2_ref):
    pos_b = posb_ref[...]
    bcol = bcol_ref[...]
    brow = brow_ref[...]
    sqc = jnp.sum(pos_b * pos_b, axis=1, keepdims=True)

    colrow = lax.broadcasted_iota(jnp.int32, (1, N), 1)
    bmin = bcol_ref[0, 0]
    bmax = bcol_ref[RB - 1, 0]
    c0 = jnp.min(jnp.where(brow == bmin, colrow, N))
    c1 = jnp.max(jnp.where(brow == bmax, colrow, -1)) + 1
    cstart = jnp.minimum((c0 >> 7) << 7, N - WLEN)
    cstart = pl.multiple_of(cstart, 128)

    @pl.when(c1 - cstart <= WLEN)
    def _fast():
        posw = posT_ref[:, pl.ds(cstart, WLEN)]
        mm = jnp.dot(pos_b, posw, preferred_element_type=jnp.float32)
        sqr_w = jnp.sum(posw * posw, axis=0, keepdims=True)
        d2 = sqc + sqr_w - 2.0 * mm
        cross = bcol != brow_ref[:, pl.ds(cstart, WLEN)]
        d2 = jnp.where(cross, jnp.float32(BIG), d2)
        pidx = lax.broadcasted_iota(jnp.int32, (RB, 128), 1)
        pval = jnp.where(pidx < jnp.minimum(c0, cstart), jnp.float32(BIG),
                         jnp.float32(jnp.inf))
        v = jnp.concatenate([pval, d2], axis=1)
        ia = jnp.concatenate(
            [pidx,
             lax.broadcasted_iota(jnp.int32, (RB, WLEN), 1) + cstart], axis=1)
        is_ = []
        for _ in range(K):
            m = jnp.min(v, axis=1, keepdims=True)
            eq = v == m
            ji = jnp.min(jnp.where(eq, ia, jnp.int32(2**30)), axis=1,
                         keepdims=True)
            v = jnp.where(eq & (ia == ji), jnp.float32(jnp.inf), v)
            is_.append(ji)
        idx_ref[...] = jnp.concatenate(is_, axis=1)

    @pl.when(c1 - cstart > WLEN)
    def _full():
        posT = posT_ref[...]
        mm = jnp.dot(pos_b, posT, preferred_element_type=jnp.float32)
        sqr = jnp.sum(posT * posT, axis=0, keepdims=True)
        d2 = sqc + sqr - 2.0 * mm
        d2 = jnp.where(bcol != brow, jnp.float32(BIG), d2)
        col = lax.broadcasted_iota(jnp.int32, (RB, N), 1)
        cols = []
        for _ in range(K):
            m = jnp.min(d2, axis=1, keepdims=True)
            ji = jnp.min(jnp.where(d2 == m, col, jnp.int32(N)), axis=1,
                         keepdims=True)
            d2 = jnp.where(col == ji, jnp.float32(jnp.inf), d2)
            cols.append(ji)
        idx_ref[...] = jnp.concatenate(cols, axis=1)

    w1p = w1p_ref[...]
    a1 = jnp.dot(pos_b, w1h_ref[...] + w1p,
                 preferred_element_type=jnp.float32)
    a1_ref[...] = jnp.concatenate(
        [a1, jnp.zeros((RB, GW - H), jnp.float32)], axis=1)
    b1_ref[...] = jnp.dot(pos_b, w1p, preferred_element_type=jnp.float32)
    b2_ref[...] = jnp.dot(pos_b, w2p_ref[...],
                          preferred_element_type=jnp.float32)


_knn_call = pl.pallas_call(
    _knn_proj_body,
    grid=(N // RB,),
    in_specs=[
        pl.BlockSpec((RB, 3), lambda i: (i, 0)),
        pl.BlockSpec((3, N), lambda i: (0, 0)),
        pl.BlockSpec((RB, 1), lambda i: (i, 0)),
        pl.BlockSpec((1, N), lambda i: (0, 0)),
        pl.BlockSpec((3, H), lambda i: (0, 0)),
        pl.BlockSpec((3, H), lambda i: (0, 0)),
        pl.BlockSpec((3, H), lambda i: (0, 0)),
    ],
    out_specs=[
        pl.BlockSpec((RB, K), lambda i: (i, 0)),
        pl.BlockSpec((RB, GW), lambda i: (i, 0)),
        pl.BlockSpec((RB, H), lambda i: (i, 0)),
        pl.BlockSpec((RB, H), lambda i: (i, 0)),
    ],
    out_shape=[
        jax.ShapeDtypeStruct((N, K), jnp.int32),
        jax.ShapeDtypeStruct((N, GW), jnp.float32),
        jax.ShapeDtypeStruct((N, H), jnp.float32),
        jax.ShapeDtypeStruct((N, H), jnp.float32),
    ],
)


_NE = N * K
_IDX_COLS = 128
_IDX_ROWS = _NE // _IDX_COLS


@functools.cache
def _make_sc_gather():
    info = plsc.get_sparse_core_info()
    nc, ns = info.num_cores, info.num_subcores
    nw = nc * ns
    rows_per_w = _NE // nw
    chunks = rows_per_w // _IDX_COLS
    rounds = 4
    cpr = chunks // rounds
    mesh = plsc.VectorSubcoreMesh(core_axis_name="c", subcore_axis_name="s")

    @functools.partial(
        pl.kernel,
        out_type=jax.ShapeDtypeStruct((_NE, H), jnp.float32),
        mesh=mesh,
        scratch_types=[
            pltpu.VMEM((chunks, _IDX_COLS), jnp.int32),
            pltpu.VMEM((cpr * _IDX_COLS, GW), jnp.float32),
            pltpu.SemaphoreType.DMA,
        ],
        compiler_params=pltpu.CompilerParams(use_tc_tiling_on_sc=False),
    )
    def gather_k(table_hbm, idx_hbm, out_hbm, idx_v, rows_v, sem):
        c = lax.axis_index("c")
        s = lax.axis_index("s")
        wid = s * nc + c
        pltpu.sync_copy(idx_hbm.at[pl.ds(wid * chunks, chunks)], idx_v)
        for r in range(rounds):
            cops = [
                pltpu.async_copy(
                    table_hbm.at[idx_v.at[r * cpr + j]],
                    rows_v.at[pl.ds(j * _IDX_COLS, _IDX_COLS)], sem)
                for j in range(cpr)
            ]
            for cop in cops:
                cop.wait()
            pltpu.sync_copy(
                rows_v.at[:, pl.ds(0, H)],
                out_hbm.at[pl.ds(wid * rows_per_w + r * cpr * _IDX_COLS,
                                 cpr * _IDX_COLS)])

    return gather_k


def _conv1_body(e1_ref, b1_ref, b2_ref, w1b_ref, b1a_ref, b1b_ref, w2h_ref,
                a2_ref):
    b1 = b1_ref[...]
    b1a = b1a_ref[...]
    w1b = w1b_ref[...]
    acc = jnp.full((R3, H), -jnp.inf, jnp.float32)
    for k in range(K):
        pre = e1_ref[k] - b1 + b1a
        acc = jnp.maximum(acc, jnp.dot(jnp.maximum(pre, 0.0), w1b,
                                       preferred_element_type=jnp.float32))
    h1 = jnp.maximum(acc + b1b_ref[...], 0.0)
    a2 = jnp.dot(h1, w2h_ref[...],
                 preferred_element_type=jnp.float32) + b2_ref[...]
    a2_ref[...] = jnp.concatenate(
        [a2, jnp.zeros((R3, GW - H), jnp.float32)], axis=1)


_conv1_call = pl.pallas_call(
    _conv1_body,
    grid=(N // R3,),
    in_specs=[
        pl.BlockSpec((K, R3, H), lambda i: (0, i, 0)),
        pl.BlockSpec((R3, H), lambda i: (i, 0)),
        pl.BlockSpec((R3, H), lambda i: (i, 0)),
        pl.BlockSpec((H, H), lambda i: (0, 0)),
        pl.BlockSpec((1, H), lambda i: (0, 0)),
        pl.BlockSpec((1, H), lambda i: (0, 0)),
        pl.BlockSpec((H, H), lambda i: (0, 0)),
    ],
    out_specs=pl.BlockSpec((R3, GW), lambda i: (i, 0)),
    out_shape=jax.ShapeDtypeStruct((N, GW), jnp.float32),
)


def _conv2_pool_body(e2_ref, b2_ref, bcol_ref, w2b_ref, b2a_ref, b2b_ref,
                     wc_ref, bcls_ref, out_ref, gacc_ref):
    i = pl.program_id(0)
    b2 = b2_ref[...]
    b2a = b2a_ref[...]
    w2b = w2b_ref[...]
    acc = jnp.full((R3, H), -jnp.inf, jnp.float32)
    for k in range(K):
        pre = e2_ref[k] - b2 + b2a
        acc = jnp.maximum(acc, jnp.dot(jnp.maximum(pre, 0.0), w2b,
                                       preferred_element_type=jnp.float32))
    h2 = jnp.maximum(acc + b2b_ref[...], 0.0)
    bcol = bcol_ref[...]
    gs = []
    for s in range(G):
        m = jnp.where(bcol == jnp.float32(s), h2, -jnp.inf)
        gs.append(jnp.max(m, axis=0, keepdims=True))
    g = jnp.concatenate(gs, axis=0)

    @pl.when(i == 0)
    def _():
        gacc_ref[...] = jnp.full((G, H), -jnp.inf, jnp.float32)

    gacc_ref[...] = jnp.maximum(gacc_ref[...], g)

    @pl.when(i == (N // R3) - 1)
    def _():
        out_ref[...] = jnp.dot(gacc_ref[...], wc_ref[...],
                               preferred_element_type=jnp.float32) + bcls_ref[...]


_conv2_call = pl.pallas_call(
    _conv2_pool_body,
    grid=(N // R3,),
    in_specs=[
        pl.BlockSpec((K, R3, H), lambda i: (0, i, 0)),
        pl.BlockSpec((R3, H), lambda i: (i, 0)),
        pl.BlockSpec((R3, 1), lambda i: (i, 0)),
        pl.BlockSpec((H, H), lambda i: (0, 0)),
        pl.BlockSpec((1, H), lambda i: (0, 0)),
        pl.BlockSpec((1, H), lambda i: (0, 0)),
        pl.BlockSpec((H, NCLS), lambda i: (0, 0)),
        pl.BlockSpec((1, NCLS), lambda i: (0, 0)),
    ],
    out_specs=pl.BlockSpec((G, NCLS), lambda i: (0, 0)),
    out_shape=jax.ShapeDtypeStruct((G, NCLS), jnp.float32),
    scratch_shapes=[pltpu.VMEM((G, H), jnp.float32)],
)


def kernel(pos, batch, W1a, b1a, W1b, b1b, W2a, b2a, W2b, b2b, Wc, bc):
    bf = batch.astype(jnp.float32)
    idx, a1, bb1, bb2 = _knn_call(pos, pos.T, bf.reshape(N, 1),
                                  bf.reshape(1, N), W1a[:3], W1a[3:],
                                  W2a[H:])
    idx2d = idx.T.reshape(_IDX_ROWS, _IDX_COLS)
    e1 = _make_sc_gather()(a1, idx2d)
    a2 = _conv1_call(e1.reshape(K, N, H), bb1, bb2, W1b, b1a.reshape(1, H),
                     b1b.reshape(1, H), W2a[:H])
    e2 = _make_sc_gather()(a2, idx2d)
    return _conv2_call(e2.reshape(K, N, H), bb2, bf.reshape(N, 1), W2b,
                       b2a.reshape(1, H), b2b.reshape(1, H), Wc,
                       bc.reshape(1, NCLS))


# speedup vs baseline: 1.1355x; 1.1355x over previous
import functools

import jax
import jax.numpy as jnp
from jax import lax
from jax.experimental import pallas as pl
from jax.experimental.pallas import tpu as pltpu
from jax.experimental.pallas import tpu_sc as plsc

N = 4096
K = 16
G = 8
H = 32
NCLS = 40
RB = 256
WLEN = 1408
R3 = 512
GW = 128
BIG = 1e30


def _knn_proj_body(posb_ref, posT_ref, bcol_ref, brow_ref, w1h_ref, w1p_ref,
                   w2p_ref, idx_ref, a1_ref, b1_ref, b2_ref):
    pos_b = posb_ref[...]
    bcol = bcol_ref[...]
    brow = brow_ref[...]
    sqc = jnp.sum(pos_b * pos_b, axis=1, keepdims=True)

    colrow = lax.broadcasted_iota(jnp.int32, (1, N), 1)
    bmin = bcol_ref[0, 0]
    bmax = bcol_ref[RB - 1, 0]
    c0 = jnp.min(jnp.where(brow == bmin, colrow, N))
    c1 = jnp.max(jnp.where(brow == bmax, colrow, -1)) + 1
    cstart = jnp.minimum((c0 >> 7) << 7, N - WLEN)
    cstart = pl.multiple_of(cstart, 128)

    @pl.when(c1 - cstart <= WLEN)
    def _fast():
        posw = posT_ref[:, pl.ds(cstart, WLEN)]
        mm = jnp.dot(pos_b, posw, preferred_element_type=jnp.float32)
        sqr_w = jnp.sum(posw * posw, axis=0, keepdims=True)
        d2 = sqc + sqr_w - 2.0 * mm
        cross = bcol != brow_ref[:, pl.ds(cstart, WLEN)]
        d2 = jnp.where(cross, jnp.float32(BIG), d2)
        pidx = lax.broadcasted_iota(jnp.int32, (RB, 128), 1)
        pval = jnp.where(pidx < jnp.minimum(c0, cstart), jnp.float32(BIG),
                         jnp.float32(jnp.inf))
        v = jnp.concatenate([pval, d2], axis=1)
        ia = jnp.concatenate(
            [pidx,
             lax.broadcasted_iota(jnp.int32, (RB, WLEN), 1) + cstart], axis=1)
        is_ = []
        for _ in range(K):
            m = jnp.min(v, axis=1, keepdims=True)
            eq = v == m
            ji = jnp.min(jnp.where(eq, ia, jnp.int32(2**30)), axis=1,
                         keepdims=True)
            v = jnp.where(eq & (ia == ji), jnp.float32(jnp.inf), v)
            is_.append(ji)
        idx_ref[...] = jnp.concatenate(is_, axis=1)

    @pl.when(c1 - cstart > WLEN)
    def _full():
        posT = posT_ref[...]
        mm = jnp.dot(pos_b, posT, preferred_element_type=jnp.float32)
        sqr = jnp.sum(posT * posT, axis=0, keepdims=True)
        d2 = sqc + sqr - 2.0 * mm
        d2 = jnp.where(bcol != brow, jnp.float32(BIG), d2)
        col = lax.broadcasted_iota(jnp.int32, (RB, N), 1)
        cols = []
        for _ in range(K):
            m = jnp.min(d2, axis=1, keepdims=True)
            ji = jnp.min(jnp.where(d2 == m, col, jnp.int32(N)), axis=1,
                         keepdims=True)
            d2 = jnp.where(col == ji, jnp.float32(jnp.inf), d2)
            cols.append(ji)
        idx_ref[...] = jnp.concatenate(cols, axis=1)

    w1p = w1p_ref[...]
    a1 = jnp.dot(pos_b, w1h_ref[...] + w1p,
                 preferred_element_type=jnp.float32)
    a1_ref[...] = jnp.concatenate(
        [a1, jnp.zeros((RB, GW - H), jnp.float32)], axis=1)
    b1_ref[...] = jnp.dot(pos_b, w1p, preferred_element_type=jnp.float32)
    b2_ref[...] = jnp.dot(pos_b, w2p_ref[...],
                          preferred_element_type=jnp.float32)


_knn_call = pl.pallas_call(
    _knn_proj_body,
    grid=(N // RB,),
    in_specs=[
        pl.BlockSpec((RB, 3), lambda i: (i, 0)),
        pl.BlockSpec((3, N), lambda i: (0, 0)),
        pl.BlockSpec((RB, 1), lambda i: (i, 0)),
        pl.BlockSpec((1, N), lambda i: (0, 0)),
        pl.BlockSpec((3, H), lambda i: (0, 0)),
        pl.BlockSpec((3, H), lambda i: (0, 0)),
        pl.BlockSpec((3, H), lambda i: (0, 0)),
    ],
    out_specs=[
        pl.BlockSpec((RB, K), lambda i: (i, 0)),
        pl.BlockSpec((RB, GW), lambda i: (i, 0)),
        pl.BlockSpec((RB, H), lambda i: (i, 0)),
        pl.BlockSpec((RB, H), lambda i: (i, 0)),
    ],
    out_shape=[
        jax.ShapeDtypeStruct((N, K), jnp.int32),
        jax.ShapeDtypeStruct((N, GW), jnp.float32),
        jax.ShapeDtypeStruct((N, H), jnp.float32),
        jax.ShapeDtypeStruct((N, H), jnp.float32),
    ],
)


_NE = N * K
_IDX_COLS = 128
_IDX_ROWS = _NE // _IDX_COLS


@functools.cache
def _make_sc_gather():
    info = plsc.get_sparse_core_info()
    nc, ns = info.num_cores, info.num_subcores
    nw = nc * ns
    rows_per_w = _NE // nw
    chunks = rows_per_w // _IDX_COLS
    rounds = 4
    cpr = chunks // rounds
    mesh = plsc.VectorSubcoreMesh(core_axis_name="c", subcore_axis_name="s")

    @functools.partial(
        pl.kernel,
        out_type=jax.ShapeDtypeStruct((_NE, GW), jnp.float32),
        mesh=mesh,
        scratch_types=[
            pltpu.VMEM((chunks, _IDX_COLS), jnp.int32),
            pltpu.VMEM((cpr * _IDX_COLS, GW), jnp.float32),
            pltpu.SemaphoreType.DMA,
        ],
    )
    def gather_k(table_hbm, idx_hbm, out_hbm, idx_v, rows_v, sem):
        c = lax.axis_index("c")
        s = lax.axis_index("s")
        wid = s * nc + c
        pltpu.sync_copy(idx_hbm.at[pl.ds(wid * chunks, chunks)], idx_v)
        for r in range(rounds):
            cops = [
                pltpu.async_copy(
                    table_hbm.at[idx_v.at[r * cpr + j]],
                    rows_v.at[pl.ds(j * _IDX_COLS, _IDX_COLS)], sem)
                for j in range(cpr)
            ]
            for cop in cops:
                cop.wait()
            pltpu.sync_copy(
                rows_v,
                out_hbm.at[pl.ds(wid * rows_per_w + r * cpr * _IDX_COLS,
                                 cpr * _IDX_COLS)])

    return gather_k


def _conv1_body(e1_ref, b1_ref, b2_ref, w1b_ref, b1a_ref, b1b_ref, w2h_ref,
                a2_ref):
    b1 = b1_ref[...]
    b1a = b1a_ref[...]
    w1b = w1b_ref[...]
    acc = jnp.full((R3, H), -jnp.inf, jnp.float32)
    for k in range(K):
        pre = e1_ref[k][:, :H] - b1 + b1a
        acc = jnp.maximum(acc, jnp.dot(jnp.maximum(pre, 0.0), w1b,
                                       preferred_element_type=jnp.float32))
    h1 = jnp.maximum(acc + b1b_ref[...], 0.0)
    a2 = jnp.dot(h1, w2h_ref[...],
                 preferred_element_type=jnp.float32) + b2_ref[...]
    a2_ref[...] = jnp.concatenate(
        [a2, jnp.zeros((R3, GW - H), jnp.float32)], axis=1)


_conv1_call = pl.pallas_call(
    _conv1_body,
    grid=(N // R3,),
    in_specs=[
        pl.BlockSpec((K, R3, GW), lambda i: (0, i, 0)),
        pl.BlockSpec((R3, H), lambda i: (i, 0)),
        pl.BlockSpec((R3, H), lambda i: (i, 0)),
        pl.BlockSpec((H, H), lambda i: (0, 0)),
        pl.BlockSpec((1, H), lambda i: (0, 0)),
        pl.BlockSpec((1, H), lambda i: (0, 0)),
        pl.BlockSpec((H, H), lambda i: (0, 0)),
    ],
    out_specs=pl.BlockSpec((R3, GW), lambda i: (i, 0)),
    out_shape=jax.ShapeDtypeStruct((N, GW), jnp.float32),
)


def _conv2_pool_body(e2_ref, b2_ref, bcol_ref, w2b_ref, b2a_ref, b2b_ref,
                     wc_ref, bcls_ref, out_ref, gacc_ref):
    i = pl.program_id(0)
    b2 = b2_ref[...]
    b2a = b2a_ref[...]
    w2b = w2b_ref[...]
    acc = jnp.full((R3, H), -jnp.inf, jnp.float32)
    for k in range(K):
        pre = e2_ref[k][:, :H] - b2 + b2a
        acc = jnp.maximum(acc, jnp.dot(jnp.maximum(pre, 0.0), w2b,
                                       preferred_element_type=jnp.float32))
    h2 = jnp.maximum(acc + b2b_ref[...], 0.0)
    bcol = bcol_ref[...]
    gs = []
    for s in range(G):
        m = jnp.where(bcol == jnp.float32(s), h2, -jnp.inf)
        gs.append(jnp.max(m, axis=0, keepdims=True))
    g = jnp.concatenate(gs, axis=0)

    @pl.when(i == 0)
    def _():
        gacc_ref[...] = jnp.full((G, H), -jnp.inf, jnp.float32)

    gacc_ref[...] = jnp.maximum(gacc_ref[...], g)

    @pl.when(i == (N // R3) - 1)
    def _():
        out_ref[...] = jnp.dot(gacc_ref[...], wc_ref[...],
                               preferred_element_type=jnp.float32) + bcls_ref[...]


_conv2_call = pl.pallas_call(
    _conv2_pool_body,
    grid=(N // R3,),
    in_specs=[
        pl.BlockSpec((K, R3, GW), lambda i: (0, i, 0)),
        pl.BlockSpec((R3, H), lambda i: (i, 0)),
        pl.BlockSpec((R3, 1), lambda i: (i, 0)),
        pl.BlockSpec((H, H), lambda i: (0, 0)),
        pl.BlockSpec((1, H), lambda i: (0, 0)),
        pl.BlockSpec((1, H), lambda i: (0, 0)),
        pl.BlockSpec((H, NCLS), lambda i: (0, 0)),
        pl.BlockSpec((1, NCLS), lambda i: (0, 0)),
    ],
    out_specs=pl.BlockSpec((G, NCLS), lambda i: (0, 0)),
    out_shape=jax.ShapeDtypeStruct((G, NCLS), jnp.float32),
    scratch_shapes=[pltpu.VMEM((G, H), jnp.float32)],
)


def kernel(pos, batch, W1a, b1a, W1b, b1b, W2a, b2a, W2b, b2b, Wc, bc):
    bf = batch.astype(jnp.float32)
    idx, a1, bb1, bb2 = _knn_call(pos, pos.T, bf.reshape(N, 1),
                                  bf.reshape(1, N), W1a[:3], W1a[3:],
                                  W2a[H:])
    idx2d = idx.T.reshape(_IDX_ROWS, _IDX_COLS)
    e1 = _make_sc_gather()(a1, idx2d)
    a2 = _conv1_call(e1.reshape(K, N, GW), bb1, bb2, W1b, b1a.reshape(1, H),
                     b1b.reshape(1, H), W2a[:H])
    e2 = _make_sc_gather()(a2, idx2d)
    return _conv2_call(e2.reshape(K, N, GW), bb2, bf.reshape(N, 1), W2b,
                       b2a.reshape(1, H), b2b.reshape(1, H), Wc,
                       bc.reshape(1, NCLS))
